# baseline (speedup 1.0000x reference)
import functools

import jax
import jax.numpy as jnp
from jax import lax
from jax.experimental import pallas as pl
from jax.experimental.pallas import tpu as pltpu
from jax.experimental.pallas import tpu_sc as plsc

N = 10000
E = 320000
D = 128
HEADS = 8
DH = D // HEADS

NC = 2
NS = 16
NW = NC * NS
EPW = E // NW
CHUNK = 80
NCHUNK = EPW // CHUNK

BE = 1000
BN = 1000
BM = 1000
BES = 2000


def _gather_body(xsrc, xdst, src, dst, gA, gB, idx_s, idx_d, bufA, bufB,
                 semA, semB):
    wid = lax.axis_index("s") * NC + lax.axis_index("c")
    base0 = wid * EPW

    def body(j, carry):
        base = base0 + j * CHUNK
        pltpu.sync_copy(src.at[pl.ds(base, CHUNK)], idx_s)
        pltpu.sync_copy(dst.at[pl.ds(base, CHUNK)], idx_d)
        ca = pltpu.async_copy(xsrc.at[idx_s], bufA, semA)
        cb = pltpu.async_copy(xdst.at[idx_d], bufB, semB)
        ca.wait()
        cb.wait()
        pltpu.sync_copy(bufA, gA.at[pl.ds(base, CHUNK)])
        pltpu.sync_copy(bufB, gB.at[pl.ds(base, CHUNK)])
        return carry

    lax.fori_loop(0, NCHUNK, body, 0)


@functools.cache
def _gather_kernel():
    return pl.kernel(
        _gather_body,
        mesh=plsc.VectorSubcoreMesh(core_axis_name="c", subcore_axis_name="s"),
        out_type=[
        jax.ShapeDtypeStruct((E, D), jnp.float32),
        jax.ShapeDtypeStruct((E, D), jnp.float32),
    ],
    scratch_types=[
        pltpu.VMEM((CHUNK,), jnp.int32),
        pltpu.VMEM((CHUNK,), jnp.int32),
        pltpu.VMEM((CHUNK, D), jnp.float32),
        pltpu.VMEM((CHUNK, D), jnp.float32),
        pltpu.SemaphoreType.DMA,
        pltpu.SemaphoreType.DMA,
    ],
    )


def _edge_block(gA, gB, ea, W0s, W0d, W0e, b0, Ac, cv, Wv1, vb1, Rr, msg, wp):
    f32 = jnp.float32
    pre = (jnp.dot(gA[...], W0s[...], preferred_element_type=f32)
           + jnp.dot(gB[...], W0d[...], preferred_element_type=f32)
           + jnp.dot(ea[...], W0e[...], preferred_element_type=f32)
           + b0[...])
    h = jnp.maximum(pre, 0.0)
    s = jnp.dot(h[:, :D], Ac[...], preferred_element_type=f32) + cv[...]
    w = jnp.exp(jnp.clip(s, -75.0, 75.0))
    v = jnp.dot(h[:, D:], Wv1[...], preferred_element_type=f32) + vb1[...]
    msg[...] = jnp.dot(w, Rr[...], preferred_element_type=f32) * v
    wp[...] = w


def _edge_call(gA, gB, ea, W0s, W0d, W0e, b0, Ac, cv, Wv1, vb1, Rr):
    grid = (E // BE,)
    full = lambda shape: pl.BlockSpec(shape, lambda i: (0,) * len(shape))
    return pl.pallas_call(
        _edge_block,
        grid=grid,
        in_specs=[
            pl.BlockSpec((BE, D), lambda i: (i, 0)),
            pl.BlockSpec((BE, D), lambda i: (i, 0)),
            pl.BlockSpec((BE, 16), lambda i: (i, 0)),
            full((D, 2 * D)), full((D, 2 * D)), full((16, 2 * D)),
            full((1, 2 * D)), full((D, DH)), full((1, DH)),
            full((D, D)), full((1, D)), full((DH, D)),
        ],
        out_specs=[
            pl.BlockSpec((BE, D), lambda i: (i, 0)),
            pl.BlockSpec((BE, DH), lambda i: (i, 0)),
        ],
        out_shape=[
            jax.ShapeDtypeStruct((E, D), jnp.float32),
            jax.ShapeDtypeStruct((E, DH), jnp.float32),
        ],
    )(gA, gB, ea, W0s, W0d, W0e, b0, Ac, cv, Wv1, vb1, Rr)


def _scatter_block(dstb, msg, wp, onum, oden):
    i = pl.program_id(0)
    j = pl.program_id(1)

    @pl.when(j == 0)
    def _():
        onum[...] = jnp.zeros_like(onum)
        oden[...] = jnp.zeros_like(oden)

    f32 = jnp.float32
    bf16 = jnp.bfloat16
    ids = lax.broadcasted_iota(jnp.int32, (BM, BES), 0) + i * BM
    dstv = dstb[...].reshape(1, BES)
    oh = (ids == dstv).astype(bf16)
    onum[...] += jnp.dot(oh, msg[...].astype(bf16), preferred_element_type=f32)
    oden[...] += jnp.dot(oh, wp[...].astype(bf16), preferred_element_type=f32)


def _scatter_call(dst2, msg, wp):
    grid = (N // BM, E // BES)
    return pl.pallas_call(
        _scatter_block,
        grid=grid,
        in_specs=[
            pl.BlockSpec((1, 1, BES), lambda i, j: (j, 0, 0)),
            pl.BlockSpec((BES, D), lambda i, j: (j, 0)),
            pl.BlockSpec((BES, DH), lambda i, j: (j, 0)),
        ],
        out_specs=[
            pl.BlockSpec((BM, D), lambda i, j: (i, 0)),
            pl.BlockSpec((BM, DH), lambda i, j: (i, 0)),
        ],
        out_shape=[
            jax.ShapeDtypeStruct((N, D), jnp.float32),
            jax.ShapeDtypeStruct((N, DH), jnp.float32),
        ],
    )(dst2, msg, wp)


def _finish_block(num_r, den_r, oW0, ob0, oW1, ob1, Rr, out):
    f32 = jnp.float32
    num = num_r[...]
    den_rep = jnp.dot(den_r[...], Rr[...], preferred_element_type=f32)
    gg = jnp.maximum(num / (den_rep + 1e-16), 0.0)
    ho = jnp.maximum(jnp.dot(gg, oW0[...], preferred_element_type=f32)
                     + ob0[...], 0.0)
    out[...] = jnp.maximum(jnp.dot(ho, oW1[...], preferred_element_type=f32)
                           + ob1[...] + ho, 0.0)


def _finish_call(num, den, oW0, ob0, oW1, ob1, Rr):
    grid = (N // BN,)
    full = lambda shape: pl.BlockSpec(shape, lambda i: (0,) * len(shape))
    return pl.pallas_call(
        _finish_block,
        grid=grid,
        in_specs=[
            pl.BlockSpec((BN, D), lambda i: (i, 0)),
            pl.BlockSpec((BN, DH), lambda i: (i, 0)),
            full((D, D)), full((1, D)), full((D, D)), full((1, D)),
            full((DH, D)),
        ],
        out_specs=pl.BlockSpec((BN, D), lambda i: (i, 0)),
        out_shape=jax.ShapeDtypeStruct((N, D), jnp.float32),
    )(num, den, oW0, ob0, oW1, ob1, Rr)


def kernel(x_src, x_dst, edge_attr, edge_index, q,
           k_W0, k_b0, k_W1, k_b1,
           v_W0, v_b0, v_W1, v_b1,
           o_W0, o_b0, o_W1, o_b1):
    src = edge_index[0]
    dst = edge_index[1]

    W0cat = jnp.concatenate([k_W0, v_W0], axis=1)
    W0s, W0d, W0e = W0cat[:D], W0cat[D:2 * D], W0cat[2 * D:]
    b0 = jnp.concatenate([k_b0, v_b0]).reshape(1, 2 * D)
    lane = jnp.arange(D)
    Qh = jnp.zeros((D, HEADS), jnp.float32).at[lane, lane // DH].set(q[0])
    Ac = (k_W1 + jnp.eye(D, dtype=jnp.float32)) @ Qh / jnp.sqrt(float(DH))
    Ac = jnp.pad(Ac, ((0, 0), (0, DH - HEADS)))
    cv = jnp.pad(k_b1 @ Qh / jnp.sqrt(float(DH)), (0, DH - HEADS))
    cv = cv.reshape(1, DH)
    Wv1 = v_W1 + jnp.eye(D, dtype=jnp.float32)
    vb1 = v_b1.reshape(1, D)
    Rr = jnp.zeros((DH, D), jnp.float32).at[lane // DH, lane].set(1.0)

    gA, gB = _gather_kernel()(x_src, x_dst, src, dst)
    msg, wp = _edge_call(gA, gB, edge_attr, W0s, W0d, W0e, b0, Ac, cv,
                         Wv1, vb1, Rr)
    dst2 = dst.reshape(E // BES, 1, BES)
    num, den = _scatter_call(dst2, msg, wp)
    return _finish_call(num, den, o_W0, o_b0.reshape(1, D),
                        o_W1, o_b1.reshape(1, D), Rr)


# speedup vs baseline: 1.0829x; 1.0829x over previous
import functools

import jax
import jax.numpy as jnp
from jax import lax
from jax.experimental import pallas as pl
from jax.experimental.pallas import tpu as pltpu
from jax.experimental.pallas import tpu_sc as plsc

N = 10000
E = 320000
D = 128
HEADS = 8
DH = D // HEADS

NC = 2
NS = 16
NW = NC * NS
EPW = E // NW
CHUNK = 80
NCHUNK = EPW // CHUNK

BE = 1000
BN = 1000
BM = 1000
BES = 2000


def _gather_body(xsrc, xdst, src, dst, gA, gB, idx_s, idx_d, bufA, bufB,
                 semA, semB):
    wid = lax.axis_index("s") * NC + lax.axis_index("c")
    base0 = wid * EPW

    def body(j, carry):
        base = base0 + j * CHUNK
        pltpu.sync_copy(src.at[pl.ds(base, CHUNK)], idx_s)
        pltpu.sync_copy(dst.at[pl.ds(base, CHUNK)], idx_d)
        ca = pltpu.async_copy(xsrc.at[idx_s], bufA, semA)
        cb = pltpu.async_copy(xdst.at[idx_d], bufB, semB)
        ca.wait()
        cb.wait()
        pltpu.sync_copy(bufA, gA.at[pl.ds(base, CHUNK)])
        pltpu.sync_copy(bufB, gB.at[pl.ds(base, CHUNK)])
        return carry

    lax.fori_loop(0, NCHUNK, body, 0)


@functools.cache
def _gather_kernel():
    return pl.kernel(
        _gather_body,
        mesh=plsc.VectorSubcoreMesh(core_axis_name="c", subcore_axis_name="s"),
        out_type=[
        jax.ShapeDtypeStruct((E, D), jnp.float32),
        jax.ShapeDtypeStruct((E, D), jnp.float32),
    ],
    scratch_types=[
        pltpu.VMEM((CHUNK,), jnp.int32),
        pltpu.VMEM((CHUNK,), jnp.int32),
        pltpu.VMEM((CHUNK, D), jnp.float32),
        pltpu.VMEM((CHUNK, D), jnp.float32),
        pltpu.SemaphoreType.DMA,
        pltpu.SemaphoreType.DMA,
    ],
    )


def _edge_block(gA, gB, ea, W0s, W0d, W0e, b0, Ac, cv, Wv1, vb1, Rr, msg, wp):
    f32 = jnp.float32
    pre = (jnp.dot(gA[...], W0s[...], preferred_element_type=f32)
           + jnp.dot(gB[...], W0d[...], preferred_element_type=f32)
           + jnp.dot(ea[...], W0e[...], preferred_element_type=f32)
           + b0[...])
    h = jnp.maximum(pre, 0.0)
    s = jnp.dot(h[:, :D], Ac[...], preferred_element_type=f32) + cv[...]
    w = jnp.exp(jnp.clip(s, -75.0, 75.0))
    v = jnp.dot(h[:, D:], Wv1[...], preferred_element_type=f32) + vb1[...]
    msg[...] = jnp.dot(w, Rr[...], preferred_element_type=f32) * v
    wp[...] = w


def _edge_call(gA, gB, ea, W0s, W0d, W0e, b0, Ac, cv, Wv1, vb1, Rr):
    grid = (E // BE,)
    full = lambda shape: pl.BlockSpec(shape, lambda i: (0,) * len(shape))
    return pl.pallas_call(
        _edge_block,
        grid=grid,
        in_specs=[
            pl.BlockSpec((BE, D), lambda i: (i, 0)),
            pl.BlockSpec((BE, D), lambda i: (i, 0)),
            pl.BlockSpec((BE, 16), lambda i: (i, 0)),
            full((D, 2 * D)), full((D, 2 * D)), full((16, 2 * D)),
            full((1, 2 * D)), full((D, DH)), full((1, DH)),
            full((D, D)), full((1, D)), full((DH, D)),
        ],
        out_specs=[
            pl.BlockSpec((BE, D), lambda i: (i, 0)),
            pl.BlockSpec((BE, DH), lambda i: (i, 0)),
        ],
        out_shape=[
            jax.ShapeDtypeStruct((E, D), jnp.float32),
            jax.ShapeDtypeStruct((E, DH), jnp.float32),
        ],
    )(gA, gB, ea, W0s, W0d, W0e, b0, Ac, cv, Wv1, vb1, Rr)


def _scatter_block(dstb, msg, wp, onum, oden):
    j = pl.program_id(0)

    @pl.when(j == 0)
    def _():
        onum[...] = jnp.zeros_like(onum)
        oden[...] = jnp.zeros_like(oden)

    f32 = jnp.float32
    dstv = dstb[...].reshape(1, BES)
    msgv = msg[...]
    wpv = wp[...]
    iota = lax.broadcasted_iota(jnp.int32, (BM, BES), 0)
    for i in range(N // BM):
        oh = (iota + i * BM == dstv).astype(f32)
        onum[i * BM:(i + 1) * BM, :] += jnp.dot(oh, msgv,
                                                preferred_element_type=f32)
        oden[i * BM:(i + 1) * BM, :] += jnp.dot(oh, wpv,
                                                preferred_element_type=f32)


def _scatter_call(dst2, msg, wp):
    grid = (E // BES,)
    return pl.pallas_call(
        _scatter_block,
        grid=grid,
        in_specs=[
            pl.BlockSpec((1, 1, BES), lambda j: (j, 0, 0)),
            pl.BlockSpec((BES, D), lambda j: (j, 0)),
            pl.BlockSpec((BES, DH), lambda j: (j, 0)),
        ],
        out_specs=[
            pl.BlockSpec((N, D), lambda j: (0, 0)),
            pl.BlockSpec((N, DH), lambda j: (0, 0)),
        ],
        out_shape=[
            jax.ShapeDtypeStruct((N, D), jnp.float32),
            jax.ShapeDtypeStruct((N, DH), jnp.float32),
        ],
    )(dst2, msg, wp)


def _finish_block(num_r, den_r, oW0, ob0, oW1, ob1, Rr, out):
    f32 = jnp.float32
    num = num_r[...]
    den_rep = jnp.dot(den_r[...], Rr[...], preferred_element_type=f32)
    gg = jnp.maximum(num / (den_rep + 1e-16), 0.0)
    ho = jnp.maximum(jnp.dot(gg, oW0[...], preferred_element_type=f32)
                     + ob0[...], 0.0)
    out[...] = jnp.maximum(jnp.dot(ho, oW1[...], preferred_element_type=f32)
                           + ob1[...] + ho, 0.0)


def _finish_call(num, den, oW0, ob0, oW1, ob1, Rr):
    grid = (N // BN,)
    full = lambda shape: pl.BlockSpec(shape, lambda i: (0,) * len(shape))
    return pl.pallas_call(
        _finish_block,
        grid=grid,
        in_specs=[
            pl.BlockSpec((BN, D), lambda i: (i, 0)),
            pl.BlockSpec((BN, DH), lambda i: (i, 0)),
            full((D, D)), full((1, D)), full((D, D)), full((1, D)),
            full((DH, D)),
        ],
        out_specs=pl.BlockSpec((BN, D), lambda i: (i, 0)),
        out_shape=jax.ShapeDtypeStruct((N, D), jnp.float32),
    )(num, den, oW0, ob0, oW1, ob1, Rr)


def kernel(x_src, x_dst, edge_attr, edge_index, q,
           k_W0, k_b0, k_W1, k_b1,
           v_W0, v_b0, v_W1, v_b1,
           o_W0, o_b0, o_W1, o_b1):
    src = edge_index[0]
    dst = edge_index[1]

    W0cat = jnp.concatenate([k_W0, v_W0], axis=1)
    W0s, W0d, W0e = W0cat[:D], W0cat[D:2 * D], W0cat[2 * D:]
    b0 = jnp.concatenate([k_b0, v_b0]).reshape(1, 2 * D)
    lane = jnp.arange(D)
    Qh = jnp.zeros((D, HEADS), jnp.float32).at[lane, lane // DH].set(q[0])
    Ac = (k_W1 + jnp.eye(D, dtype=jnp.float32)) @ Qh / jnp.sqrt(float(DH))
    Ac = jnp.pad(Ac, ((0, 0), (0, DH - HEADS)))
    cv = jnp.pad(k_b1 @ Qh / jnp.sqrt(float(DH)), (0, DH - HEADS))
    cv = cv.reshape(1, DH)
    Wv1 = v_W1 + jnp.eye(D, dtype=jnp.float32)
    vb1 = v_b1.reshape(1, D)
    Rr = jnp.zeros((DH, D), jnp.float32).at[lane // DH, lane].set(1.0)

    gA, gB = _gather_kernel()(x_src, x_dst, src, dst)
    msg, wp = _edge_call(gA, gB, edge_attr, W0s, W0d, W0e, b0, Ac, cv,
                         Wv1, vb1, Rr)
    dst2 = dst.reshape(E // BES, 1, BES)
    num, den = _scatter_call(dst2, msg, wp)
    return _finish_call(num, den, o_W0, o_b0.reshape(1, D),
                        o_W1, o_b1.reshape(1, D), Rr)
